# baseline (speedup 1.0000x reference)
import functools

import jax
import jax.numpy as jnp
from jax import lax
from jax.experimental import pallas as pl
from jax.experimental.pallas import tpu as pltpu
from jax.experimental.pallas import tpu_sc as plsc

_INFO = plsc.get_sparse_core_info()
_NC = _INFO.num_cores
_NS = _INFO.num_subcores
_NW = _NC * _NS

_CHUNK = 2048
_NBUF = 2


@functools.partial(jax.jit, static_argnums=(2, 3))
def _gather(idx_t, table, batch, hist):
    D = table.shape[1]
    n_hgrp = 8
    n_bq = _NW // n_hgrp
    h_per_grp = hist // n_hgrp
    b_per_q = batch // n_bq
    halves = b_per_q // _CHUNK
    mesh = plsc.VectorSubcoreMesh(core_axis_name="c", subcore_axis_name="s")

    @functools.partial(
        pl.kernel,
        mesh=mesh,
        out_type=jax.ShapeDtypeStruct((hist, D, batch, 1), jnp.float32),
        scratch_types=[
            pltpu.VMEM((_NBUF, _CHUNK), jnp.int32),
            pltpu.VMEM((_NBUF, _CHUNK, D), jnp.float32),
            pltpu.SemaphoreType.DMA((_NBUF,)),
            pltpu.SemaphoreType.DMA((_NBUF,)),
        ],
        compiler_params=pltpu.CompilerParams(use_tc_tiling_on_sc=False),
    )
    def k(idx_hbm, table_hbm, out_hbm, idx_v, rows_v, gsem, osem):
        wid = lax.axis_index("s") * _NC + lax.axis_index("c")
        h_base = (wid // n_bq) * h_per_grp
        b_base = (wid % n_bq) * b_per_q

        def idx_in(g, s):
            off = (h_base + g) * batch + b_base + s * _CHUNK
            pltpu.sync_copy(idx_hbm.at[pl.ds(off, _CHUNK)], idx_v.at[s])

        def gather(s):
            return pltpu.make_async_copy(
                table_hbm.at[idx_v.at[s]], rows_v.at[s], gsem.at[s])

        def col_out(g, s, j):
            b0 = b_base + s * _CHUNK
            return pltpu.make_async_copy(
                rows_v.at[s, pl.ds(0, _CHUNK), pl.ds(j, 1)],
                out_hbm.at[h_base + g, j, pl.ds(b0, _CHUNK), pl.ds(0, 1)],
                osem.at[s])

        def outs_start(g, s):
            for j in range(D):
                col_out(g, s, j).start()

        def outs_wait(g, s):
            for j in range(D):
                col_out(g, s, j).wait()

        for s in range(_NBUF):
            idx_in(0, s)
            gather(s).start()
            if s >= 1:
                gather(s - 1).wait()
                outs_start(0, s - 1)

        def step(g, carry):
            for s in range(_NBUF):
                outs_wait(g - 1, s)
                idx_in(g, s)
                gather(s).start()
                sp = s - 1 if s >= 1 else _NBUF - 1
                gp = g if s >= 1 else g - 1
                gather(sp).wait()
                outs_start(gp, sp)
            return carry

        lax.fori_loop(1, h_per_grp, step, 0)

        gather(_NBUF - 1).wait()
        outs_start(h_per_grp - 1, _NBUF - 1)
        for s in range(_NBUF):
            outs_wait(h_per_grp - 1, s)

    return k(idx_t, table)


def kernel(input_ids, table):
    Bt, H = input_ids.shape
    D = table.shape[1]
    idx_t = input_ids.T.reshape(-1).astype(jnp.int32)
    out_t = _gather(idx_t, table, Bt, H)
    return out_t.reshape(H, D, Bt).transpose(2, 0, 1)


# speedup vs baseline: 77.9989x; 77.9989x over previous
import functools

import jax
import jax.numpy as jnp
from jax import lax
from jax.experimental import pallas as pl
from jax.experimental.pallas import tpu as pltpu
from jax.experimental.pallas import tpu_sc as plsc

_INFO = plsc.get_sparse_core_info()
_NC = _INFO.num_cores
_NS = _INFO.num_subcores
_NW = _NC * _NS
_L = _INFO.num_lanes

_CHUNK = 1024
_NBUF = 2


@functools.partial(jax.jit, static_argnums=(2, 3))
def _gather_t(idx_t, table, batch, hist):
    D = table.shape[1]
    n_hgrp = 8
    n_bq = _NW // n_hgrp
    h_per_grp = hist // n_hgrp
    b_per_q = batch // n_bq
    bchunks = b_per_q // _CHUNK
    nchunks = h_per_grp * bchunks
    mesh = plsc.VectorSubcoreMesh(core_axis_name="c", subcore_axis_name="s")

    @functools.partial(
        pl.kernel,
        mesh=mesh,
        out_type=jax.ShapeDtypeStruct((hist, D, batch), jnp.float32),
        scratch_types=[
            pltpu.VMEM((_NBUF, _CHUNK), jnp.int32),
            pltpu.VMEM((_NBUF, _CHUNK, D), jnp.float32),
            pltpu.VMEM((_NBUF, D, _CHUNK), jnp.float32),
            pltpu.SemaphoreType.DMA((_NBUF,)),
            pltpu.SemaphoreType.DMA((_NBUF,)),
        ],
        compiler_params=pltpu.CompilerParams(
            use_tc_tiling_on_sc=False, needs_layout_passes=False),
    )
    def k(idx_hbm, table_hbm, out_hbm, idx_v, rows_v, cols_v, gsem, osem):
        wid = lax.axis_index("s") * _NC + lax.axis_index("c")
        h_base = (wid // n_bq) * h_per_grp
        b_base = (wid % n_bq) * b_per_q

        def chunk_hb(c):
            h = h_base + c // bchunks
            b0 = b_base + (c % bchunks) * _CHUNK
            return h, b0

        def idx_in(c, s):
            h, b0 = chunk_hb(c)
            pltpu.sync_copy(idx_hbm.at[pl.ds(h * batch + b0, _CHUNK)],
                            idx_v.at[s])

        def gather(s):
            return pltpu.make_async_copy(
                table_hbm.at[idx_v.at[s]], rows_v.at[s], gsem.at[s])

        def out(c, s):
            h, b0 = chunk_hb(c)
            return pltpu.make_async_copy(
                cols_v.at[s], out_hbm.at[h, :, pl.ds(b0, _CHUNK)], osem.at[s])

        lane = lax.iota(jnp.int32, _L)

        def transpose(s):
            def body(i0, carry):
                row_ids = i0 * _L + lane
                for j in range(D):
                    col_ids = jnp.full((_L,), j, jnp.int32)
                    v = plsc.load_gather(rows_v.at[s], [row_ids, col_ids])
                    cols_v[s, j, pl.ds(i0 * _L, _L)] = v
                return carry
            lax.fori_loop(0, _CHUNK // _L, body, 0)

        for s in range(_NBUF):
            idx_in(s, s)
            gather(s).start()
            if s >= 1:
                gather(s - 1).wait()
                transpose(s - 1)
                out(s - 1, s - 1).start()

        def group(g, carry):
            for s in range(_NBUF):
                c = g * _NBUF + s
                out(c - 2 * _NBUF + _NBUF, s).wait()
                idx_in(c, s)
                gather(s).start()
                sp = s - 1 if s >= 1 else _NBUF - 1
                gather(sp).wait()
                transpose(sp)
                out(c - 1, sp).start()
            return carry

        lax.fori_loop(1, nchunks // _NBUF, group, 0)

        last = nchunks - 1
        gather(_NBUF - 1).wait()
        transpose(_NBUF - 1)
        out(last, _NBUF - 1).start()
        for s in range(_NBUF):
            out(last - (_NBUF - 1) + s, s).wait()

    return k(idx_t, table)


def kernel(input_ids, table):
    Bt, H = input_ids.shape
    D = table.shape[1]
    idx_t = input_ids.T.reshape(-1).astype(jnp.int32)
    out_t = _gather_t(idx_t, table, Bt, H)
    return out_t.transpose(2, 0, 1)
